# Initial kernel scaffold; baseline (speedup 1.0000x reference)
import jax
import jax.numpy as jnp
from jax.experimental import pallas as pl


def kernel(src_node_ids, dst_node_ids, RP, lambda_weights, W1, b1, W2, b2):
    raise NotImplementedError("write your pallas kernel here")


# speedup vs baseline: 2.6880x; 2.6880x over previous
import functools

import jax
import jax.numpy as jnp
from jax import lax
from jax.experimental import pallas as pl
from jax.experimental.pallas import tpu as pltpu
from jax.experimental.pallas import tpu_sc as plsc

_NC = 2
_NS = 16
_L = 16


def _sc_gather_combine(table, src_i, dst_i, lam16, m, k1, node_num, dim, b):
    nw = _NC * _NS
    bpw = b // nw
    tk = m * k1
    kd = k1 * dim
    chunk = 64
    nch = bpw // chunk

    mesh = plsc.VectorSubcoreMesh(core_axis_name="c", subcore_axis_name="s")

    @functools.partial(
        pl.kernel,
        out_type=(
            jax.ShapeDtypeStruct((b, kd), jnp.float32),
            jax.ShapeDtypeStruct((b, kd), jnp.float32),
        ),
        mesh=mesh,
        scratch_types=[
            pltpu.VMEM((bpw,), jnp.int32),
            pltpu.VMEM((tk, chunk), jnp.int32),
            pltpu.VMEM((tk, chunk, dim), jnp.float32),
            pltpu.VMEM((chunk, kd), jnp.float32),
            pltpu.VMEM((_L,), jnp.float32),
            pltpu.VMEM((_L,), jnp.float32),
            pltpu.VMEM((2 * _L,), jnp.float32),
            pltpu.SemaphoreType.DMA,
        ],
        compiler_params=pltpu.CompilerParams(needs_layout_passes=False),
    )
    def sc_kern(table_h, src_h, dst_h, lam_h, out_s, out_d,
                ids_v, idx_v, gbuf, acc, lam_v, sm_v, wt_v, sem):
        wid = lax.axis_index("s") * _NC + lax.axis_index("c")
        base = wid * bpw

        pltpu.sync_copy(lam_h, lam_v)
        lam = lam_v[...]
        lanes = lax.broadcasted_iota(jnp.int32, (_L,), 0)
        gbase = (lanes // m) * m
        offs = [plsc.load_gather(lam_v, [gbase + mm]) for mm in range(m)]
        gmax = offs[0]
        for o in offs[1:]:
            gmax = jnp.maximum(gmax, o)
        sm_v[...] = jnp.exp(lam - gmax)
        den = plsc.load_gather(sm_v, [gbase])
        for mm in range(1, m):
            den = den + plsc.load_gather(sm_v, [gbase + mm])
        wt_v[pl.ds(0, _L)] = sm_v[...] / den
        wt_v[pl.ds(_L, _L)] = sm_v[...] / den
        wvecs = [None] * tk
        for k in range(k1):
            for mm in range(m):
                lane = jnp.full((_L,), _L + k * m + mm, jnp.int32)
                wvecs[mm * k1 + k] = plsc.load_gather(wt_v, [lane])

        for ids_h, out_h in ((src_h, out_s), (dst_h, out_d)):
            pltpu.sync_copy(ids_h.at[pl.ds(base, bpw)], ids_v)
            for c in range(nch):
                def build(s, carry):
                    ids16 = ids_v[pl.ds(c * chunk + s * _L, _L)]
                    for j in range(tk):
                        idx_v[j, pl.ds(s * _L, _L)] = ids16 + j * node_num
                    return carry
                lax.fori_loop(0, chunk // _L, build, 0)

                descs = [
                    pltpu.async_copy(table_h.at[idx_v.at[j]], gbuf.at[j], sem)
                    for j in range(tk)
                ]
                for d in descs:
                    d.wait()

                def comb(r, carry):
                    for k in range(k1):
                        for s in range(dim // _L):
                            tot = wvecs[k] * gbuf[k, r, pl.ds(s * _L, _L)]
                            for mm in range(1, m):
                                tot = tot + wvecs[mm * k1 + k] * gbuf[mm * k1 + k, r, pl.ds(s * _L, _L)]
                            acc[r, pl.ds(k * dim + s * _L, _L)] = tot
                    return carry
                lax.fori_loop(0, chunk, comb, 0)

                pltpu.sync_copy(acc, out_h.at[pl.ds(base + c * chunk, chunk)])

    return sc_kern(table, src_i, dst_i, lam16)


def _gram_mlp(rs, rd, w1, b1, w2, b2, b, k1, dim, pair):
    bb = 1024
    nsix = 2 * k1
    hid = w1.shape[1]

    def body(rs_ref, rd_ref, w1_ref, b1_ref, w2_ref, b2_ref, o_ref):
        xs = [rs_ref[:, i * dim:(i + 1) * dim] for i in range(k1)]
        xs += [rd_ref[:, i * dim:(i + 1) * dim] for i in range(k1)]
        prods = {}
        cols = []
        for i in range(nsix):
            for j in range(nsix):
                if (j, i) in prods:
                    f = prods[(j, i)]
                else:
                    f = jnp.sum(xs[i] * xs[j], axis=1, keepdims=True)
                    prods[(i, j)] = f
                cols.append(f)
        feat = jnp.concatenate(cols, axis=1)
        h = jnp.dot(feat, w1_ref[...], preferred_element_type=jnp.float32)
        h = jnp.maximum(h + b1_ref[...], 0.0)
        o = jnp.dot(h, w2_ref[...], preferred_element_type=jnp.float32)
        o_ref[...] = o + b2_ref[...]

    return pl.pallas_call(
        body,
        grid=(b // bb,),
        in_specs=[
            pl.BlockSpec((bb, k1 * dim), lambda i: (i, 0)),
            pl.BlockSpec((bb, k1 * dim), lambda i: (i, 0)),
            pl.BlockSpec((pair, hid), lambda i: (0, 0)),
            pl.BlockSpec((1, hid), lambda i: (0, 0)),
            pl.BlockSpec((hid, pair), lambda i: (0, 0)),
            pl.BlockSpec((1, pair), lambda i: (0, 0)),
        ],
        out_specs=pl.BlockSpec((bb, pair), lambda i: (i, 0)),
        out_shape=jax.ShapeDtypeStruct((b, pair), jnp.float32),
    )(rs, rd, w1, b1, w2, b2)


def kernel(src_node_ids, dst_node_ids, RP, lambda_weights, W1, b1, W2, b2):
    m, k1, node_num, dim = RP.shape
    b = src_node_ids.shape[0]
    pair = (2 * k1) ** 2

    table = RP.reshape(m * k1 * node_num, dim)
    src_i = src_node_ids.astype(jnp.int32)
    dst_i = dst_node_ids.astype(jnp.int32)
    lam16 = jnp.zeros((_L,), jnp.float32).at[: k1 * m].set(
        lambda_weights.reshape(-1).astype(jnp.float32))

    rs, rd = _sc_gather_combine(table, src_i, dst_i, lam16,
                                m, k1, node_num, dim, b)
    return _gram_mlp(rs, rd, W1.astype(jnp.float32),
                     b1.reshape(1, -1).astype(jnp.float32),
                     W2.astype(jnp.float32),
                     b2.reshape(1, -1).astype(jnp.float32),
                     b, k1, dim, pair)


# speedup vs baseline: 3.1901x; 1.1868x over previous
import functools

import jax
import jax.numpy as jnp
from jax import lax
from jax.experimental import pallas as pl
from jax.experimental.pallas import tpu as pltpu
from jax.experimental.pallas import tpu_sc as plsc

_NC = 2
_NS = 16
_L = 16


def _sc_gather_combine(table, src_i, dst_i, lam16, m, k1, node_num, dim, b):
    nw = _NC * _NS
    bpw = b // nw
    tk = m * k1
    kd = k1 * dim
    chunk = 64
    nch = bpw // chunk

    mesh = plsc.VectorSubcoreMesh(core_axis_name="c", subcore_axis_name="s")

    @functools.partial(
        pl.kernel,
        out_type=(
            jax.ShapeDtypeStruct((b, kd), jnp.float32),
            jax.ShapeDtypeStruct((b, kd), jnp.float32),
        ),
        mesh=mesh,
        scratch_types=[
            pltpu.VMEM((2, bpw), jnp.int32),
            pltpu.VMEM((2 * m, chunk), jnp.int32),
            pltpu.VMEM((2 * m, chunk, dim), jnp.float32),
            pltpu.VMEM((chunk, kd), jnp.float32),
            pltpu.VMEM((_L,), jnp.float32),
            pltpu.VMEM((_L,), jnp.float32),
            pltpu.VMEM((2 * _L,), jnp.float32),
            pltpu.SemaphoreType.DMA,
            pltpu.SemaphoreType.DMA,
        ],
        compiler_params=pltpu.CompilerParams(needs_layout_passes=False),
    )
    def sc_kern(table_h, src_h, dst_h, lam_h, out_s, out_d,
                ids_v, idx_v, gbuf, acc, lam_v, sm_v, wt_v, sem0, sem1):
        wid = lax.axis_index("s") * _NC + lax.axis_index("c")
        base = wid * bpw

        pltpu.sync_copy(lam_h, lam_v)
        lam = lam_v[...]
        lanes = lax.broadcasted_iota(jnp.int32, (_L,), 0)
        gbase = (lanes // m) * m
        offs = [plsc.load_gather(lam_v, [gbase + mm]) for mm in range(m)]
        gmax = offs[0]
        for o in offs[1:]:
            gmax = jnp.maximum(gmax, o)
        sm_v[...] = jnp.exp(lam - gmax)
        den = plsc.load_gather(sm_v, [gbase])
        for mm in range(1, m):
            den = den + plsc.load_gather(sm_v, [gbase + mm])
        wt_v[pl.ds(0, _L)] = sm_v[...] / den
        wt_v[pl.ds(_L, _L)] = sm_v[...] / den
        wvecs = [None] * tk
        for k in range(k1):
            for mm in range(m):
                lane = jnp.full((_L,), _L + k * m + mm, jnp.int32)
                wvecs[mm * k1 + k] = plsc.load_gather(wt_v, [lane])

        pltpu.sync_copy(src_h.at[pl.ds(base, bpw)], ids_v.at[0])
        pltpu.sync_copy(dst_h.at[pl.ds(base, bpw)], ids_v.at[1])

        units = [(side, c, k)
                 for side in range(2) for c in range(nch) for k in range(k1)]
        sems = (sem0, sem1)
        descs = {}

        def fire(u):
            side, c, k = units[u]
            bank = u % 2
            for s in range(chunk // _L):
                ids16 = ids_v[side, pl.ds(c * chunk + s * _L, _L)]
                for mm in range(m):
                    idx_v[bank * m + mm, pl.ds(s * _L, _L)] = (
                        ids16 + (mm * k1 + k) * node_num)
            descs[u] = [
                pltpu.async_copy(table_h.at[idx_v.at[bank * m + mm]],
                                 gbuf.at[bank * m + mm], sems[bank])
                for mm in range(m)
            ]

        fire(0)
        for u in range(len(units)):
            side, c, k = units[u]
            bank = u % 2
            if u + 1 < len(units):
                fire(u + 1)
            for d in descs.pop(u):
                d.wait()

            def comb(r, carry):
                for s in range(dim // _L):
                    tot = wvecs[k] * gbuf[bank * m, r, pl.ds(s * _L, _L)]
                    for mm in range(1, m):
                        tot = tot + wvecs[mm * k1 + k] * gbuf[bank * m + mm, r, pl.ds(s * _L, _L)]
                    acc[r, pl.ds(k * dim + s * _L, _L)] = tot
                return carry
            lax.fori_loop(0, chunk, comb, 0)

            if k == k1 - 1:
                out_h = out_s if side == 0 else out_d
                pltpu.sync_copy(acc, out_h.at[pl.ds(base + c * chunk, chunk)])

    return sc_kern(table, src_i, dst_i, lam16)


def _gram_mlp(rs, rd, w1, b1, w2, b2, b, k1, dim, pair):
    bb = 1024
    nsix = 2 * k1
    hid = w1.shape[1]

    def body(rs_ref, rd_ref, w1_ref, b1_ref, w2_ref, b2_ref, o_ref):
        xs = [rs_ref[:, i * dim:(i + 1) * dim] for i in range(k1)]
        xs += [rd_ref[:, i * dim:(i + 1) * dim] for i in range(k1)]
        prods = {}
        cols = []
        for i in range(nsix):
            for j in range(nsix):
                if (j, i) in prods:
                    f = prods[(j, i)]
                else:
                    f = jnp.sum(xs[i] * xs[j], axis=1, keepdims=True)
                    prods[(i, j)] = f
                cols.append(f)
        feat = jnp.concatenate(cols, axis=1)
        h = jnp.dot(feat, w1_ref[...], preferred_element_type=jnp.float32)
        h = jnp.maximum(h + b1_ref[...], 0.0)
        o = jnp.dot(h, w2_ref[...], preferred_element_type=jnp.float32)
        o_ref[...] = o + b2_ref[...]

    return pl.pallas_call(
        body,
        grid=(b // bb,),
        in_specs=[
            pl.BlockSpec((bb, k1 * dim), lambda i: (i, 0)),
            pl.BlockSpec((bb, k1 * dim), lambda i: (i, 0)),
            pl.BlockSpec((pair, hid), lambda i: (0, 0)),
            pl.BlockSpec((1, hid), lambda i: (0, 0)),
            pl.BlockSpec((hid, pair), lambda i: (0, 0)),
            pl.BlockSpec((1, pair), lambda i: (0, 0)),
        ],
        out_specs=pl.BlockSpec((bb, pair), lambda i: (i, 0)),
        out_shape=jax.ShapeDtypeStruct((b, pair), jnp.float32),
    )(rs, rd, w1, b1, w2, b2)


def kernel(src_node_ids, dst_node_ids, RP, lambda_weights, W1, b1, W2, b2):
    m, k1, node_num, dim = RP.shape
    b = src_node_ids.shape[0]
    pair = (2 * k1) ** 2

    table = RP.reshape(m * k1 * node_num, dim)
    src_i = src_node_ids.astype(jnp.int32)
    dst_i = dst_node_ids.astype(jnp.int32)
    lam16 = jnp.zeros((_L,), jnp.float32).at[: k1 * m].set(
        lambda_weights.reshape(-1).astype(jnp.float32))

    rs, rd = _sc_gather_combine(table, src_i, dst_i, lam16,
                                m, k1, node_num, dim, b)
    return _gram_mlp(rs, rd, W1.astype(jnp.float32),
                     b1.reshape(1, -1).astype(jnp.float32),
                     W2.astype(jnp.float32),
                     b2.reshape(1, -1).astype(jnp.float32),
                     b, k1, dim, pair)


# speedup vs baseline: 5.8567x; 1.8359x over previous
import functools

import jax
import jax.numpy as jnp
from jax import lax
from jax.experimental import pallas as pl
from jax.experimental.pallas import tpu as pltpu
from jax.experimental.pallas import tpu_sc as plsc

_NC = 2
_NS = 16
_L = 16


def _sc_gather_combine(table, src_i, dst_i, lam16, m, k1, node_num, dim, b):
    nw = _NC * _NS
    bpw = b // nw
    tk = m * k1
    chunk = 128
    nch = bpw // chunk

    mesh = plsc.VectorSubcoreMesh(core_axis_name="c", subcore_axis_name="s")

    @functools.partial(
        pl.kernel,
        out_type=(
            jax.ShapeDtypeStruct((k1, b, dim), jnp.float32),
            jax.ShapeDtypeStruct((k1, b, dim), jnp.float32),
        ),
        mesh=mesh,
        scratch_types=[
            pltpu.VMEM((2, bpw), jnp.int32),
            pltpu.VMEM((2 * m, chunk), jnp.int32),
            pltpu.VMEM((2 * m, chunk, dim), jnp.float32),
            pltpu.VMEM((_L,), jnp.float32),
            pltpu.VMEM((_L,), jnp.float32),
            pltpu.VMEM((2 * _L,), jnp.float32),
            pltpu.SemaphoreType.DMA,
            pltpu.SemaphoreType.DMA,
            pltpu.SemaphoreType.DMA,
            pltpu.SemaphoreType.DMA,
        ],
        compiler_params=pltpu.CompilerParams(needs_layout_passes=False),
    )
    def sc_kern(table_h, src_h, dst_h, lam_h, out_s, out_d,
                ids_v, idx_v, gbuf, lam_v, sm_v, wt_v, sem0, sem1, osem0, osem1):
        wid = lax.axis_index("s") * _NC + lax.axis_index("c")
        base = wid * bpw

        pltpu.sync_copy(lam_h, lam_v)
        lam = lam_v[...]
        lanes = lax.broadcasted_iota(jnp.int32, (_L,), 0)
        gbase = (lanes // m) * m
        offs = [plsc.load_gather(lam_v, [gbase + mm]) for mm in range(m)]
        gmax = offs[0]
        for o in offs[1:]:
            gmax = jnp.maximum(gmax, o)
        sm_v[...] = jnp.exp(lam - gmax)
        den = plsc.load_gather(sm_v, [gbase])
        for mm in range(1, m):
            den = den + plsc.load_gather(sm_v, [gbase + mm])
        wt_v[pl.ds(0, _L)] = sm_v[...] / den
        wt_v[pl.ds(_L, _L)] = sm_v[...] / den
        wvecs = [None] * tk
        for k in range(k1):
            for mm in range(m):
                lane = jnp.full((_L,), _L + k * m + mm, jnp.int32)
                wvecs[mm * k1 + k] = plsc.load_gather(wt_v, [lane])

        pltpu.sync_copy(src_h.at[pl.ds(base, bpw)], ids_v.at[0])
        pltpu.sync_copy(dst_h.at[pl.ds(base, bpw)], ids_v.at[1])

        units = [(side, c, k)
                 for side in range(2) for c in range(nch) for k in range(k1)]
        sems = (sem0, sem1)
        osems = (osem0, osem1)
        descs = {}
        sdescs = {}

        def fire(u):
            side, c, k = units[u]
            bank = u % 2
            if u - 2 >= 0:
                sdescs.pop(u - 2).wait()
            for s in range(chunk // _L):
                ids16 = ids_v[side, pl.ds(c * chunk + s * _L, _L)]
                for mm in range(m):
                    idx_v[bank * m + mm, pl.ds(s * _L, _L)] = (
                        ids16 + (mm * k1 + k) * node_num)
            descs[u] = [
                pltpu.async_copy(table_h.at[idx_v.at[bank * m + mm]],
                                 gbuf.at[bank * m + mm], sems[bank])
                for mm in range(m)
            ]

        fire(0)
        for u in range(len(units)):
            side, c, k = units[u]
            bank = u % 2
            if u + 1 < len(units):
                fire(u + 1)
            for d in descs.pop(u):
                d.wait()

            def comb(r, carry):
                for s in range(dim // _L):
                    tot = wvecs[k] * gbuf[bank * m, r, pl.ds(s * _L, _L)]
                    for mm in range(1, m):
                        tot = tot + wvecs[mm * k1 + k] * gbuf[bank * m + mm, r, pl.ds(s * _L, _L)]
                    gbuf[bank * m, r, pl.ds(s * _L, _L)] = tot
                return carry
            lax.fori_loop(0, chunk, comb, 0)

            out_h = out_s if side == 0 else out_d
            sdescs[u] = pltpu.async_copy(
                gbuf.at[bank * m],
                out_h.at[k, pl.ds(base + c * chunk, chunk)], osems[bank])
        for u in sorted(sdescs):
            sdescs.pop(u).wait()

    return sc_kern(table, src_i, dst_i, lam16)


def _gram_mlp(rs, rd, w1, b1, w2, b2, b, k1, dim, pair):
    bb = 1024
    nsix = 2 * k1
    hid = w1.shape[1]

    def body(rs_ref, rd_ref, w1_ref, b1_ref, w2_ref, b2_ref, o_ref):
        xs = [rs_ref[i] for i in range(k1)]
        xs += [rd_ref[i] for i in range(k1)]
        prods = {}
        cols = []
        for i in range(nsix):
            for j in range(nsix):
                if (j, i) in prods:
                    f = prods[(j, i)]
                else:
                    f = jnp.sum(xs[i] * xs[j], axis=1, keepdims=True)
                    prods[(i, j)] = f
                cols.append(f)
        feat = jnp.concatenate(cols, axis=1)
        h = jnp.dot(feat, w1_ref[...], preferred_element_type=jnp.float32)
        h = jnp.maximum(h + b1_ref[...], 0.0)
        o = jnp.dot(h, w2_ref[...], preferred_element_type=jnp.float32)
        o_ref[...] = o + b2_ref[...]

    return pl.pallas_call(
        body,
        grid=(b // bb,),
        in_specs=[
            pl.BlockSpec((k1, bb, dim), lambda i: (0, i, 0)),
            pl.BlockSpec((k1, bb, dim), lambda i: (0, i, 0)),
            pl.BlockSpec((pair, hid), lambda i: (0, 0)),
            pl.BlockSpec((1, hid), lambda i: (0, 0)),
            pl.BlockSpec((hid, pair), lambda i: (0, 0)),
            pl.BlockSpec((1, pair), lambda i: (0, 0)),
        ],
        out_specs=pl.BlockSpec((bb, pair), lambda i: (i, 0)),
        out_shape=jax.ShapeDtypeStruct((b, pair), jnp.float32),
    )(rs, rd, w1, b1, w2, b2)


def kernel(src_node_ids, dst_node_ids, RP, lambda_weights, W1, b1, W2, b2):
    m, k1, node_num, dim = RP.shape
    b = src_node_ids.shape[0]
    pair = (2 * k1) ** 2

    table = RP.reshape(m * k1 * node_num, dim)
    src_i = src_node_ids.astype(jnp.int32)
    dst_i = dst_node_ids.astype(jnp.int32)
    lam16 = jnp.zeros((_L,), jnp.float32).at[: k1 * m].set(
        lambda_weights.reshape(-1).astype(jnp.float32))

    rs, rd = _sc_gather_combine(table, src_i, dst_i, lam16,
                                m, k1, node_num, dim, b)
    return _gram_mlp(rs, rd, W1.astype(jnp.float32),
                     b1.reshape(1, -1).astype(jnp.float32),
                     W2.astype(jnp.float32),
                     b2.reshape(1, -1).astype(jnp.float32),
                     b, k1, dim, pair)


# speedup vs baseline: 5.8750x; 1.0031x over previous
import functools

import jax
import jax.numpy as jnp
from jax import lax
from jax.experimental import pallas as pl
from jax.experimental.pallas import tpu as pltpu
from jax.experimental.pallas import tpu_sc as plsc

_NC = 2
_NS = 16
_L = 16


def _sc_gather_combine(table, src_i, dst_i, lam16, m, k1, node_num, dim, b):
    nw = _NC * _NS
    bpw = b // nw
    tk = m * k1
    chunk = 128
    nch = bpw // chunk

    mesh = plsc.VectorSubcoreMesh(core_axis_name="c", subcore_axis_name="s")

    @functools.partial(
        pl.kernel,
        out_type=(
            jax.ShapeDtypeStruct((k1, b, dim), jnp.float32),
            jax.ShapeDtypeStruct((k1, b, dim), jnp.float32),
        ),
        mesh=mesh,
        scratch_types=[
            pltpu.VMEM((2, bpw), jnp.int32),
            pltpu.VMEM((2 * m, chunk), jnp.int32),
            pltpu.VMEM((2 * m, chunk, dim), jnp.float32),
            pltpu.VMEM((_L,), jnp.float32),
            pltpu.VMEM((_L,), jnp.float32),
            pltpu.VMEM((2 * _L,), jnp.float32),
            pltpu.SemaphoreType.DMA,
            pltpu.SemaphoreType.DMA,
            pltpu.SemaphoreType.DMA,
            pltpu.SemaphoreType.DMA,
        ],
        compiler_params=pltpu.CompilerParams(needs_layout_passes=False),
    )
    def sc_kern(table_h, src_h, dst_h, lam_h, out_s, out_d,
                ids_v, idx_v, gbuf, lam_v, sm_v, wt_v, sem0, sem1, osem0, osem1):
        wid = lax.axis_index("s") * _NC + lax.axis_index("c")
        base = wid * bpw

        pltpu.sync_copy(lam_h, lam_v)
        lam = lam_v[...]
        lanes = lax.broadcasted_iota(jnp.int32, (_L,), 0)
        gbase = (lanes // m) * m
        offs = [plsc.load_gather(lam_v, [gbase + mm]) for mm in range(m)]
        gmax = offs[0]
        for o in offs[1:]:
            gmax = jnp.maximum(gmax, o)
        sm_v[...] = jnp.exp(lam - gmax)
        den = plsc.load_gather(sm_v, [gbase])
        for mm in range(1, m):
            den = den + plsc.load_gather(sm_v, [gbase + mm])
        wt_v[pl.ds(0, _L)] = sm_v[...] / den
        wt_v[pl.ds(_L, _L)] = sm_v[...] / den
        wvecs = [None] * tk
        for k in range(k1):
            for mm in range(m):
                lane = jnp.full((_L,), _L + k * m + mm, jnp.int32)
                wvecs[mm * k1 + k] = plsc.load_gather(wt_v, [lane])

        pltpu.sync_copy(src_h.at[pl.ds(base, bpw)], ids_v.at[0])
        pltpu.sync_copy(dst_h.at[pl.ds(base, bpw)], ids_v.at[1])

        units = [(side, c, k)
                 for side in range(2) for c in range(nch) for k in range(k1)]
        sems = (sem0, sem1)
        osems = (osem0, osem1)
        descs = {}
        sdescs = {}

        def fire(u):
            side, c, k = units[u]
            bank = u % 2
            if u - 2 >= 0:
                sdescs.pop(u - 2).wait()
            for s in range(chunk // _L):
                ids16 = ids_v[side, pl.ds(c * chunk + s * _L, _L)]
                for mm in range(m):
                    idx_v[bank * m + mm, pl.ds(s * _L, _L)] = (
                        ids16 + (mm * k1 + k) * node_num)
            descs[u] = [
                pltpu.async_copy(table_h.at[idx_v.at[bank * m + mm]],
                                 gbuf.at[bank * m + mm], sems[bank])
                for mm in range(m)
            ]

        fire(0)
        for u in range(len(units)):
            side, c, k = units[u]
            bank = u % 2
            if u + 1 < len(units):
                fire(u + 1)
            for d in descs.pop(u):
                d.wait()

            def comb(r, carry):
                for s in range(dim // _L):
                    tot = wvecs[k] * gbuf[bank * m, r, pl.ds(s * _L, _L)]
                    for mm in range(1, m):
                        tot = tot + wvecs[mm * k1 + k] * gbuf[bank * m + mm, r, pl.ds(s * _L, _L)]
                    gbuf[bank * m, r, pl.ds(s * _L, _L)] = tot
                return carry
            lax.fori_loop(0, chunk, comb, 0)

            out_h = out_s if side == 0 else out_d
            sdescs[u] = pltpu.async_copy(
                gbuf.at[bank * m],
                out_h.at[k, pl.ds(base + c * chunk, chunk)], osems[bank])
        for u in sorted(sdescs):
            sdescs.pop(u).wait()

    return sc_kern(table, src_i, dst_i, lam16)


def _gram_mlp(rs, rd, w1, b1, w2, b2, b, k1, dim, pair):
    bb = 2048
    nsix = 2 * k1
    hid = w1.shape[1]

    def body(rs_ref, rd_ref, w1_ref, b1_ref, w2_ref, b2_ref, o_ref):
        xs = [rs_ref[i] for i in range(k1)]
        xs += [rd_ref[i] for i in range(k1)]
        prods = {}
        cols = []
        for i in range(nsix):
            for j in range(nsix):
                if (j, i) in prods:
                    f = prods[(j, i)]
                else:
                    f = jnp.sum(xs[i] * xs[j], axis=1, keepdims=True)
                    prods[(i, j)] = f
                cols.append(f)
        feat = jnp.concatenate(cols, axis=1)
        h = jnp.dot(feat, w1_ref[...], preferred_element_type=jnp.float32)
        h = jnp.maximum(h + b1_ref[...], 0.0)
        o = jnp.dot(h, w2_ref[...], preferred_element_type=jnp.float32)
        o_ref[...] = o + b2_ref[...]

    return pl.pallas_call(
        body,
        grid=(b // bb,),
        in_specs=[
            pl.BlockSpec((k1, bb, dim), lambda i: (0, i, 0)),
            pl.BlockSpec((k1, bb, dim), lambda i: (0, i, 0)),
            pl.BlockSpec((pair, hid), lambda i: (0, 0)),
            pl.BlockSpec((1, hid), lambda i: (0, 0)),
            pl.BlockSpec((hid, pair), lambda i: (0, 0)),
            pl.BlockSpec((1, pair), lambda i: (0, 0)),
        ],
        out_specs=pl.BlockSpec((bb, pair), lambda i: (i, 0)),
        out_shape=jax.ShapeDtypeStruct((b, pair), jnp.float32),
    )(rs, rd, w1, b1, w2, b2)


def kernel(src_node_ids, dst_node_ids, RP, lambda_weights, W1, b1, W2, b2):
    m, k1, node_num, dim = RP.shape
    b = src_node_ids.shape[0]
    pair = (2 * k1) ** 2

    table = RP.reshape(m * k1 * node_num, dim)
    src_i = src_node_ids.astype(jnp.int32)
    dst_i = dst_node_ids.astype(jnp.int32)
    lam16 = jnp.zeros((_L,), jnp.float32).at[: k1 * m].set(
        lambda_weights.reshape(-1).astype(jnp.float32))

    rs, rd = _sc_gather_combine(table, src_i, dst_i, lam16,
                                m, k1, node_num, dim, b)
    return _gram_mlp(rs, rd, W1.astype(jnp.float32),
                     b1.reshape(1, -1).astype(jnp.float32),
                     W2.astype(jnp.float32),
                     b2.reshape(1, -1).astype(jnp.float32),
                     b, k1, dim, pair)


# speedup vs baseline: 6.2763x; 1.0683x over previous
import functools

import jax
import jax.numpy as jnp
from jax import lax
from jax.experimental import pallas as pl
from jax.experimental.pallas import tpu as pltpu
from jax.experimental.pallas import tpu_sc as plsc

_NC = 2
_NS = 16
_L = 16


def _sc_gather_combine(table, src_i, dst_i, lam16, m, k1, node_num, dim, b):
    nw = _NC * _NS
    bpw = b // nw
    tk = m * k1
    chunk = 128
    nch = bpw // chunk

    mesh = plsc.VectorSubcoreMesh(core_axis_name="c", subcore_axis_name="s")

    @functools.partial(
        pl.kernel,
        out_type=(
            jax.ShapeDtypeStruct((k1, b, dim), jnp.float32),
            jax.ShapeDtypeStruct((k1, b, dim), jnp.float32),
        ),
        mesh=mesh,
        scratch_types=[
            pltpu.VMEM((2, bpw), jnp.int32),
            pltpu.VMEM((2 * m, chunk), jnp.int32),
            pltpu.VMEM((2 * m, chunk, dim), jnp.float32),
            pltpu.VMEM((_L,), jnp.float32),
            pltpu.VMEM((_L,), jnp.float32),
            pltpu.VMEM((2 * _L,), jnp.float32),
            pltpu.SemaphoreType.DMA,
            pltpu.SemaphoreType.DMA,
            pltpu.SemaphoreType.DMA,
            pltpu.SemaphoreType.DMA,
        ],
        compiler_params=pltpu.CompilerParams(needs_layout_passes=False),
    )
    def sc_kern(table_h, src_h, dst_h, lam_h, out_s, out_d,
                ids_v, idx_v, gbuf, lam_v, sm_v, wt_v, sem0, sem1, osem0, osem1):
        wid = lax.axis_index("s") * _NC + lax.axis_index("c")
        base = wid * bpw

        pltpu.sync_copy(lam_h, lam_v)
        lam = lam_v[...]
        lanes = lax.broadcasted_iota(jnp.int32, (_L,), 0)
        gbase = (lanes // m) * m
        offs = [plsc.load_gather(lam_v, [gbase + mm]) for mm in range(m)]
        gmax = offs[0]
        for o in offs[1:]:
            gmax = jnp.maximum(gmax, o)
        sm_v[...] = jnp.exp(lam - gmax)
        den = plsc.load_gather(sm_v, [gbase])
        for mm in range(1, m):
            den = den + plsc.load_gather(sm_v, [gbase + mm])
        wt_v[pl.ds(0, _L)] = sm_v[...] / den
        wt_v[pl.ds(_L, _L)] = sm_v[...] / den
        wvecs = [None] * tk
        for k in range(k1):
            for mm in range(m):
                lane = jnp.full((_L,), _L + k * m + mm, jnp.int32)
                wvecs[mm * k1 + k] = plsc.load_gather(wt_v, [lane])

        pltpu.sync_copy(src_h.at[pl.ds(base, bpw)], ids_v.at[0])
        pltpu.sync_copy(dst_h.at[pl.ds(base, bpw)], ids_v.at[1])

        units = [(side, c, k)
                 for side in range(2) for c in range(nch) for k in range(k1)]
        sems = (sem0, sem1)
        osems = (osem0, osem1)
        descs = {}
        sdescs = {}

        def fire(u):
            side, c, k = units[u]
            bank = u % 2
            if u - 2 >= 0:
                sdescs.pop(u - 2).wait()
            for s in range(chunk // _L):
                ids16 = ids_v[side, pl.ds(c * chunk + s * _L, _L)]
                for mm in range(m):
                    idx_v[bank * m + mm, pl.ds(s * _L, _L)] = (
                        ids16 + (mm * k1 + k) * node_num)
            descs[u] = [
                pltpu.async_copy(table_h.at[idx_v.at[bank * m + mm]],
                                 gbuf.at[bank * m + mm], sems[bank])
                for mm in range(m)
            ]

        fire(0)
        for u in range(len(units)):
            side, c, k = units[u]
            bank = u % 2
            if u + 1 < len(units):
                fire(u + 1)
            for d in descs.pop(u):
                d.wait()

            def comb(r, carry):
                for s in range(dim // _L):
                    tot = wvecs[k] * gbuf[bank * m, r, pl.ds(s * _L, _L)]
                    for mm in range(1, m):
                        tot = tot + wvecs[mm * k1 + k] * gbuf[bank * m + mm, r, pl.ds(s * _L, _L)]
                    gbuf[bank * m, r, pl.ds(s * _L, _L)] = tot
                return carry
            lax.fori_loop(0, chunk, comb, 0)

            out_h = out_s if side == 0 else out_d
            sdescs[u] = pltpu.async_copy(
                gbuf.at[bank * m],
                out_h.at[k, pl.ds(base + c * chunk, chunk)], osems[bank])
        for u in sorted(sdescs):
            sdescs.pop(u).wait()

    return sc_kern(table, src_i, dst_i, lam16)


def _gram_mlp(rs, rd, w1, b1, w2, b2, b, k1, dim, pair):
    bb = 2048
    nsix = 2 * k1
    hid = w1.shape[1]

    def body(rs_ref, rd_ref, w1_ref, b1_ref, w2_ref, b2_ref, o_ref):
        xs = [rs_ref[i] for i in range(k1)]
        xs += [rd_ref[i] for i in range(k1)]
        prods = {}
        cols = []
        for i in range(nsix):
            for j in range(nsix):
                if (j, i) in prods:
                    f = prods[(j, i)]
                else:
                    f = jnp.sum(xs[i] * xs[j], axis=1, keepdims=True)
                    prods[(i, j)] = f
                cols.append(f)
        feat = jnp.concatenate(cols, axis=1)
        h = jnp.dot(feat, w1_ref[...], preferred_element_type=jnp.float32)
        h = jnp.maximum(h + b1_ref[...], 0.0)
        o = jnp.dot(h, w2_ref[...], preferred_element_type=jnp.float32)
        o_ref[...] = o + b2_ref[...]

    return pl.pallas_call(
        body,
        grid=(b // bb,),
        in_specs=[
            pl.BlockSpec((k1, bb, dim), lambda i: (0, i, 0)),
            pl.BlockSpec((k1, bb, dim), lambda i: (0, i, 0)),
            pl.BlockSpec((pair, hid), lambda i: (0, 0)),
            pl.BlockSpec((1, hid), lambda i: (0, 0)),
            pl.BlockSpec((hid, pair), lambda i: (0, 0)),
            pl.BlockSpec((1, pair), lambda i: (0, 0)),
        ],
        out_specs=pl.BlockSpec((bb, pair), lambda i: (i, 0)),
        out_shape=jax.ShapeDtypeStruct((b, pair), jnp.float32),
    )(rs, rd, w1, b1, w2, b2)


def kernel(src_node_ids, dst_node_ids, RP, lambda_weights, W1, b1, W2, b2):
    m, k1, node_num, dim = RP.shape
    b = src_node_ids.shape[0]
    pair = (2 * k1) ** 2

    table = RP.reshape(m * k1 * node_num, dim)
    src_i = src_node_ids.astype(jnp.int32)
    dst_i = dst_node_ids.astype(jnp.int32)
    lam16 = jnp.zeros((_L,), jnp.float32).at[: k1 * m].set(
        lambda_weights.reshape(-1).astype(jnp.float32))

    w1 = W1.astype(jnp.float32)
    b1r = b1.reshape(1, -1).astype(jnp.float32)
    w2 = W2.astype(jnp.float32)
    b2r = b2.reshape(1, -1).astype(jnp.float32)

    h = b // 2
    rs1, rd1 = _sc_gather_combine(table, src_i[:h], dst_i[:h], lam16,
                                  m, k1, node_num, dim, h)
    rs2, rd2 = _sc_gather_combine(table, src_i[h:], dst_i[h:], lam16,
                                  m, k1, node_num, dim, h)
    o1 = _gram_mlp(rs1, rd1, w1, b1r, w2, b2r, h, k1, dim, pair)
    o2 = _gram_mlp(rs2, rd2, w1, b1r, w2, b2r, h, k1, dim, pair)
    return jnp.concatenate([o1, o2], axis=0)
